# Initial kernel scaffold; baseline (speedup 1.0000x reference)
import jax
import jax.numpy as jnp
from jax.experimental import pallas as pl


def kernel(x_0, x_1, neighborhood_0_to_0, att):
    raise NotImplementedError("write your pallas kernel here")


# speedup vs baseline: 2.9979x; 2.9979x over previous
import functools

import jax
import jax.numpy as jnp
from jax import lax
from jax.experimental import pallas as pl
from jax.experimental.pallas import tpu as pltpu
from jax.experimental.pallas import tpu_sc as plsc

N_NODES = 10000
N_EDGES = 320000
D_FEAT = 128
K_HEADS = 3

NC = 2
NS = 16
L = 16
NW = NC * NS
EPW = N_EDGES // NW
CH = 2000
NCH = EPW // CH
GROUPS = CH // L


def _scores_kernel(wt_ref, x_ref, s_ref):
    s_ref[...] = lax.dot_general(
        wt_ref[...], x_ref[...], (((1,), (1,)), ((), ())),
        preferred_element_type=jnp.float32,
        precision=lax.Precision.HIGHEST,
    )


def _edge_kernel(s_hbm, src_hbm, tgt_hbm, heads_hbm, s_v, src_v, tgt_v, h_v):
    wid = lax.axis_index("s") * NC + lax.axis_index("c")
    base = wid * EPW
    pltpu.sync_copy(s_hbm, s_v)
    lane = lax.broadcasted_iota(jnp.int32, (L,), 0)

    def chunk_body(ci, carry):
        cbase = base + ci * CH
        pltpu.sync_copy(src_hbm.at[pl.ds(cbase, CH)], src_v)
        pltpu.sync_copy(tgt_hbm.at[pl.ds(cbase, CH)], tgt_v)

        def group_body(g, gcarry):
            gb = g * L
            s16 = src_v[pl.ds(gb, L)]
            t16 = tgt_v[pl.ds(gb, L)]
            oidx = lane * 4 + gb * 4
            for k in range(K_HEADS):
                a = plsc.load_gather(s_v, [s16 + (k * N_NODES)])
                b = plsc.load_gather(s_v, [t16 + ((3 + k) * N_NODES)])
                plsc.store_scatter(h_v, [oidx + k], jnp.maximum(a + b, 0.0))
            return gcarry

        lax.fori_loop(0, GROUPS, group_body, 0)
        pltpu.sync_copy(h_v, heads_hbm.at[pl.ds(cbase * 4, CH * 4)])
        return carry

    lax.fori_loop(0, NCH, chunk_body, 0)


def _concat_kernel(h_ref, x1_ref, o_ref):
    o_ref[...] = jnp.concatenate([h_ref[:, :K_HEADS], x1_ref[...]], axis=1)


@jax.jit
def kernel(x_0, x_1, neighborhood_0_to_0, att):
    src = neighborhood_0_to_0[0].astype(jnp.int32)
    tgt = neighborhood_0_to_0[1].astype(jnp.int32)
    wt = jnp.concatenate([att[:, :D_FEAT, 0], att[:, D_FEAT:, 0]], axis=0)
    wt8 = jnp.pad(wt, ((0, 2), (0, 0)))

    s8 = pl.pallas_call(
        _scores_kernel,
        out_shape=jax.ShapeDtypeStruct((8, N_NODES), jnp.float32),
    )(wt8, x_0)

    heads_flat = pl.kernel(
        _edge_kernel,
        out_type=jax.ShapeDtypeStruct((N_EDGES * 4,), jnp.float32),
        mesh=plsc.VectorSubcoreMesh(
            core_axis_name="c", subcore_axis_name="s",
            num_cores=NC, num_subcores=NS,
        ),
        scratch_types=[
            pltpu.VMEM((8 * N_NODES,), jnp.float32),
            pltpu.VMEM((CH,), jnp.int32),
            pltpu.VMEM((CH,), jnp.int32),
            pltpu.VMEM((CH * 4,), jnp.float32),
        ],
        compiler_params=pltpu.CompilerParams(needs_layout_passes=False),
    )(s8.reshape(8 * N_NODES), src, tgt)

    BE = 3200
    out = pl.pallas_call(
        _concat_kernel,
        grid=(N_EDGES // BE,),
        in_specs=[
            pl.BlockSpec((BE, 4), lambda i: (i, 0)),
            pl.BlockSpec((BE, 16), lambda i: (i, 0)),
        ],
        out_specs=pl.BlockSpec((BE, K_HEADS + 16), lambda i: (i, 0)),
        out_shape=jax.ShapeDtypeStruct((N_EDGES, K_HEADS + 16), jnp.float32),
    )(heads_flat.reshape(N_EDGES, 4), x_1)
    return out
